# Initial kernel scaffold; baseline (speedup 1.0000x reference)
import jax
import jax.numpy as jnp
from jax.experimental import pallas as pl


def kernel(input_ids, turns, poly_coeffs, pos_table, ln_gamma, ln_beta):
    raise NotImplementedError("write your pallas kernel here")


# speedup vs baseline: 1.3341x; 1.3341x over previous
import functools

import jax
import jax.numpy as jnp
from jax import lax
from jax.experimental import pallas as pl
from jax.experimental.pallas import tpu as pltpu
from jax.experimental.pallas import tpu_sc as plsc

_IDX_CHUNK = 128


def _make_sc_gather(vocab, n_turns, n_tok):
    info = plsc.get_sparse_core_info()
    nw = info.num_cores * info.num_subcores
    tok_per_w = n_tok // nw
    n_chunks = tok_per_w // _IDX_CHUNK
    mesh = plsc.VectorSubcoreMesh(core_axis_name="c", subcore_axis_name="s")

    @functools.partial(
        pl.kernel,
        mesh=mesh,
        compiler_params=pltpu.CompilerParams(use_tc_tiling_on_sc=False),
        out_type=jax.ShapeDtypeStruct((n_tok, n_turns), jnp.float32),
        scratch_types=[
            pltpu.VMEM((n_chunks, _IDX_CHUNK), jnp.int32),
            pltpu.VMEM((tok_per_w, n_turns), jnp.float32),
            pltpu.SemaphoreType.DMA,
        ],
    )
    def gather_kernel(turns_hbm, ids_hbm, out_hbm, idx_v, rows_v, sem):
        wid = lax.axis_index("s") * info.num_cores + lax.axis_index("c")
        base = wid * tok_per_w
        pltpu.sync_copy(ids_hbm.at[pl.ds(wid * n_chunks, n_chunks)], idx_v)
        copies = [
            pltpu.async_copy(
                turns_hbm.at[idx_v.at[c]],
                rows_v.at[pl.ds(c * _IDX_CHUNK, _IDX_CHUNK)],
                sem,
            )
            for c in range(n_chunks)
        ]
        for cp in copies:
            cp.wait()
        pltpu.sync_copy(rows_v, out_hbm.at[pl.ds(base, tok_per_w)])

    return gather_kernel


def _tc_body(x_ref, pc_ref, pos_ref, g_ref, b_ref, o_ref):
    x = x_ref[0]
    pc = pc_ref[...]
    t = x.shape[-1]
    x2 = x * x
    x3 = x2 * x
    x4 = x2 * x2
    c0 = jnp.sum(pc[0:t], axis=0, keepdims=True)
    emb = jnp.dot(x, pc[t : 2 * t], preferred_element_type=jnp.float32)
    emb = emb + jnp.dot(x2, pc[2 * t : 3 * t], preferred_element_type=jnp.float32)
    emb = emb + jnp.dot(x3, pc[3 * t : 4 * t], preferred_element_type=jnp.float32)
    emb = emb + jnp.dot(x4, pc[4 * t : 5 * t], preferred_element_type=jnp.float32)
    emb = emb + c0 + pos_ref[...]
    mean = jnp.mean(emb, axis=-1, keepdims=True)
    cen = emb - mean
    var = jnp.mean(cen * cen, axis=-1, keepdims=True)
    o_ref[0] = cen * lax.rsqrt(var + 1e-12) * g_ref[...] + b_ref[...]


def _tc_dense(x, pc, pos_table, gamma, beta):
    b, s, t = x.shape
    d = pos_table.shape[-1]
    return pl.pallas_call(
        _tc_body,
        grid=(b,),
        in_specs=[
            pl.BlockSpec((1, s, t), lambda i: (i, 0, 0)),
            pl.BlockSpec((pc.shape[0], d), lambda i: (0, 0)),
            pl.BlockSpec((s, d), lambda i: (0, 0)),
            pl.BlockSpec((1, d), lambda i: (0, 0)),
            pl.BlockSpec((1, d), lambda i: (0, 0)),
        ],
        out_specs=pl.BlockSpec((1, s, d), lambda i: (i, 0, 0)),
        out_shape=jax.ShapeDtypeStruct((b, s, d), jnp.float32),
    )(x, pc, pos_table, gamma, beta)


def kernel(input_ids, turns, poly_coeffs, pos_table, ln_gamma, ln_beta):
    b, s = input_ids.shape
    vocab, n_turns = turns.shape
    n_tok = b * s
    ids = input_ids.astype(jnp.int32).reshape(n_tok // _IDX_CHUNK, _IDX_CHUNK)
    x = _make_sc_gather(vocab, n_turns, n_tok)(turns, ids)
    pc = jnp.transpose(poly_coeffs, (1, 0, 2)).reshape(-1, poly_coeffs.shape[-1])
    out = _tc_dense(
        x.reshape(b, s, n_turns),
        pc,
        pos_table,
        ln_gamma.reshape(1, -1),
        ln_beta.reshape(1, -1),
    )
    return out
